# Initial kernel scaffold; baseline (speedup 1.0000x reference)
import jax
import jax.numpy as jnp
from jax.experimental import pallas as pl


def kernel(x, edge_index, batch, graph_features, W0, b0, g0, be0, W1, b1, g1, be1, W2, b2, g2, be2, Wl, bl):
    raise NotImplementedError("write your pallas kernel here")


# speedup vs baseline: 7.2807x; 7.2807x over previous
import functools

import jax
import jax.numpy as jnp
from jax import lax
from jax.experimental import pallas as pl
from jax.experimental.pallas import tpu as pltpu
from jax.experimental.pallas import tpu_sc as plsc

N = 10000
E = 320000
D = 128
H = 64
HW = 128
GF = 16
B = 64
C = 2
EPS = 1e-5

NC = 2
NS = 16
NW = NC * NS
CHUNK = 128
CPW = 80
EPW = CPW * CHUNK
EP = NW * EPW
PAD_NODE = N
NP = 10240
RPT = NP // NS


def _sc_mesh():
    return plsc.VectorSubcoreMesh(
        core_axis_name="c", subcore_axis_name="s",
        num_cores=NC, num_subcores=NS)


@functools.cache
def _sc_deg_kernel():
    return functools.partial(
        pl.kernel,
        out_type=jax.ShapeDtypeStruct((NC, NP), jnp.float32),
        mesh=_sc_mesh(),
        scratch_types=[
            pltpu.VMEM((CPW, CHUNK), jnp.int32),
            pltpu.VMEM((CHUNK,), jnp.float32),
            pltpu.VMEM_SHARED((NP,), jnp.float32),
        ],
    )(_sc_deg_body)


def _sc_deg_body(dst_hbm, zero1_hbm, out_hbm, dst_v, ones_v, acc_sh):
    c = lax.axis_index("c")
    s = lax.axis_index("s")
    wid = s * NC + c
    lo = s * RPT
    pltpu.sync_copy(zero1_hbm.at[pl.ds(lo, RPT)], acc_sh.at[pl.ds(lo, RPT)])
    pltpu.sync_copy(dst_hbm.at[wid], dst_v)
    for i in range(CHUNK // 16):
        ones_v[pl.ds(i * 16, 16)] = jnp.ones((16,), jnp.float32)
    plsc.subcore_barrier()

    def body(j, carry):
        pltpu.sync_copy(ones_v, acc_sh.at[dst_v.at[j]], add=True)
        return carry

    lax.fori_loop(0, CPW, body, 0)
    plsc.subcore_barrier()
    pltpu.sync_copy(acc_sh.at[pl.ds(lo, RPT)], out_hbm.at[c, pl.ds(lo, RPT)])


@functools.cache
def _sc_edge_kernel():
    return functools.partial(
        pl.kernel,
        out_type=jax.ShapeDtypeStruct((NC, NP, HW), jnp.float32),
        mesh=_sc_mesh(),
        scratch_types=[
            pltpu.VMEM((CPW, CHUNK), jnp.int32),
            pltpu.VMEM((CPW, CHUNK), jnp.int32),
            pltpu.VMEM((CHUNK, HW), jnp.float32),
            pltpu.VMEM_SHARED((NP, HW), jnp.float32),
            pltpu.SemaphoreType.DMA,
        ],
    )(_sc_edge_body)


def _sc_edge_body(xs_hbm, src_hbm, dst_hbm, zero_hbm, out_hbm,
                  src_v, dst_v, rows_v, acc_sh, sem):
    c = lax.axis_index("c")
    s = lax.axis_index("s")
    wid = s * NC + c
    lo = s * RPT
    pltpu.sync_copy(zero_hbm.at[pl.ds(lo, RPT)], acc_sh.at[pl.ds(lo, RPT)])
    pltpu.sync_copy(src_hbm.at[wid], src_v)
    pltpu.sync_copy(dst_hbm.at[wid], dst_v)
    plsc.subcore_barrier()

    def body(j, carry):
        pltpu.async_copy(xs_hbm.at[src_v.at[j]], rows_v, sem).wait()
        pltpu.sync_copy(rows_v, acc_sh.at[dst_v.at[j]], add=True)
        return carry

    lax.fori_loop(0, CPW, body, 0)
    plsc.subcore_barrier()
    pltpu.sync_copy(acc_sh.at[pl.ds(lo, RPT)], out_hbm.at[c, pl.ds(lo, RPT)])


def _tc_xw0_body(x_ref, w_ref, out_ref):
    out_ref[...] = jnp.dot(x_ref[...], w_ref[...],
                           preferred_element_type=jnp.float32)


def _tc_prep_body(degp_ref, xw_ref, dinv_ref, xs_ref):
    deg = degp_ref[:, 0:1] + degp_ref[:, 1:2] + 1.0
    dinv = jnp.where(deg > 0, lax.rsqrt(deg), 0.0)
    dinv_ref[...] = dinv
    xs_ref[...] = xw_ref[...] * dinv


def _bn_relu_mask(agg, g, be):
    rmask = lax.broadcasted_iota(jnp.int32, (NP, 1), 0) < N
    aggm = jnp.where(rmask, agg, 0.0)
    mu = jnp.sum(aggm, axis=0, keepdims=True) / N
    dcen = jnp.where(rmask, agg - mu, 0.0)
    var = jnp.sum(dcen * dcen, axis=0, keepdims=True) / N
    h = (agg - mu) * lax.rsqrt(var + EPS) * g + be
    return jnp.where(rmask, jnp.maximum(h, 0.0), 0.0)


def _tc_mid_body(p_ref, xs_ref, dinv_ref, b_ref, g_ref, be_ref, w_ref,
                 out_ref):
    scat = p_ref[0] + p_ref[1] + xs_ref[...]
    agg = scat * dinv_ref[...] + b_ref[...]
    h = _bn_relu_mask(agg, g_ref[...], be_ref[...])
    out_ref[...] = jnp.dot(h, w_ref[...],
                           preferred_element_type=jnp.float32) * dinv_ref[...]


def _tc_final_body(p_ref, xs_ref, dinv_ref, b_ref, g_ref, be_ref,
                   bt_ref, gf_ref, wl_ref, bl_ref, out_ref):
    scat = p_ref[0] + p_ref[1] + xs_ref[...]
    agg = scat * dinv_ref[...] + b_ref[...]
    h = _bn_relu_mask(agg, g_ref[...], be_ref[...])
    oh = (lax.broadcasted_iota(jnp.int32, (B, NP), 0)
          == bt_ref[...]).astype(jnp.float32)
    sums = jnp.dot(oh, h, preferred_element_type=jnp.float32)
    counts = jnp.sum(oh, axis=1, keepdims=True)
    pooled = sums / jnp.maximum(counts, 1.0)
    fused = jnp.concatenate([pooled[:, :H], gf_ref[...]], axis=1)
    out_ref[...] = jnp.dot(fused, wl_ref[...],
                           preferred_element_type=jnp.float32) + bl_ref[...]


def _tc_call(body, out_shapes):
    return pl.pallas_call(body, out_shape=out_shapes)


def _padw(w, rows, cols):
    return jnp.zeros((rows, cols), jnp.float32).at[:w.shape[0],
                                                   :w.shape[1]].set(w)


def kernel(x, edge_index, batch, graph_features,
           W0, b0, g0, be0, W1, b1, g1, be1, W2, b2, g2, be2, Wl, bl):
    f32 = jnp.float32
    x_pad = jnp.zeros((NP, D), f32).at[:N].set(x)
    pad_ids = jnp.full((EP - E,), PAD_NODE, jnp.int32)
    src = jnp.concatenate([edge_index[0], pad_ids]).reshape(NW, CPW, CHUNK)
    dst = jnp.concatenate([edge_index[1], pad_ids]).reshape(NW, CPW, CHUNK)
    bt = jnp.full((NP,), B, jnp.int32).at[:N].set(batch).reshape(1, NP)
    zero1 = jnp.zeros((NP,), f32)
    zero = jnp.zeros((NP, HW), f32)
    W0p, W1p, W2p = _padw(W0, D, HW), _padw(W1, HW, HW), _padw(W2, HW, HW)
    b0r, g0r, be0r = (_padw(v.reshape(1, H), 1, HW) for v in (b0, g0, be0))
    b1r, g1r, be1r = (_padw(v.reshape(1, H), 1, HW) for v in (b1, g1, be1))
    b2r, g2r, be2r = (_padw(v.reshape(1, H), 1, HW) for v in (b2, g2, be2))
    blr = bl.reshape(1, C)

    degp = _sc_deg_kernel()(dst, zero1)
    xw0 = _tc_call(_tc_xw0_body,
                   jax.ShapeDtypeStruct((NP, HW), f32))(x_pad, W0p)
    degp_t = degp.T
    dinv, xs0 = _tc_call(_tc_prep_body,
                         (jax.ShapeDtypeStruct((NP, 1), f32),
                          jax.ShapeDtypeStruct((NP, HW), f32)))(degp_t, xw0)

    p0 = _sc_edge_kernel()(xs0, src, dst, zero)
    xs1 = _tc_call(_tc_mid_body, jax.ShapeDtypeStruct((NP, HW), f32))(
        p0, xs0, dinv, b0r, g0r, be0r, W1p)
    p1 = _sc_edge_kernel()(xs1, src, dst, zero)
    xs2 = _tc_call(_tc_mid_body, jax.ShapeDtypeStruct((NP, HW), f32))(
        p1, xs1, dinv, b1r, g1r, be1r, W2p)
    p2 = _sc_edge_kernel()(xs2, src, dst, zero)
    out = _tc_call(_tc_final_body, jax.ShapeDtypeStruct((B, C), f32))(
        p2, xs2, dinv, b2r, g2r, be2r, bt, graph_features, Wl, blr)
    return out
